# Initial kernel scaffold; baseline (speedup 1.0000x reference)
import jax
import jax.numpy as jnp
from jax.experimental import pallas as pl


def kernel(labels, x, W_word, W_label):
    raise NotImplementedError("write your pallas kernel here")


# speedup vs baseline: 2.1079x; 2.1079x over previous
import jax
import jax.numpy as jnp
from jax import lax
from jax.experimental import pallas as pl
from jax.experimental.pallas import tpu as pltpu
from jax.experimental.pallas import tpu_sc as plsc

BATCH = 1024
LM1 = 49
VOCAB = 1000
VPAD = 1024
N_CLS = 100
NC = 2
NS = 16
NW = NC * NS
B_PER_W = BATCH // NW
NCT = VPAD // 128
GROUP_OFFS = (0, 8, 16, 24, 32, 40)


def _fill_buf(buf, bufc):
    for k in range(8):
        for c in range(7):
            for m in range(8):
                buf[k, pl.ds(128 * c + 16 * m, 16)] = bufc[c, k,
                                                          pl.ds(16 * m, 16)]
        buf[k, pl.ds(984, 16)] = bufc[7, k, pl.ds(88, 16)]
        for m in range(6):
            buf[k, pl.ds(896 + 16 * m, 16)] = bufc[7, k, pl.ds(16 * m, 16)]


def _sc_body(idxe_hbm, wall_hbm, out_hbm, idxe_v, bufc_a, bufc_b, buf_a,
             buf_b, gsem_a, gsem_b, wsem_a, wsem_b):
    wid = lax.axis_index("s") * NC + lax.axis_index("c")
    base = wid * B_PER_W

    pltpu.sync_copy(idxe_hbm.at[pl.ds(base * NCT, B_PER_W * NCT)], idxe_v)

    bufcs = (bufc_a, bufc_b)
    bufs = (buf_a, buf_b)
    gsems = (gsem_a, gsem_b)
    wsems = (wsem_a, wsem_b)

    def issue_writes(buf, wsem, b):
        for off in GROUP_OFFS:
            pltpu.async_copy(buf, out_hbm.at[b, pl.ds(off, 8), :], wsem)
        pltpu.async_copy(buf.at[pl.ds(0, 1), :],
                         out_hbm.at[b, pl.ds(48, 1), :], wsem)

    def drain_writes(buf, wsem):
        for _ in GROUP_OFFS:
            pltpu.make_async_copy(buf, out_hbm.at[base, pl.ds(0, 8), :],
                                  wsem).wait()
        pltpu.make_async_copy(buf.at[pl.ds(0, 1), :],
                              out_hbm.at[base, pl.ds(48, 1), :], wsem).wait()

    def do_elem(i, slot):
        bufc, buf = bufcs[slot], bufs[slot]
        gsem, wsem = gsems[slot], wsems[slot]
        off = pl.multiple_of(i * NCT, NCT)
        pltpu.async_copy(wall_hbm.at[idxe_v.at[pl.ds(off, NCT)]], bufc,
                         gsem).wait()
        _fill_buf(buf, bufc)
        issue_writes(buf, wsem, base + i)

    def pair(p, _):
        i0 = 2 * p

        @pl.when(p > 0)
        def _drain():
            drain_writes(buf_a, wsem_a)
            drain_writes(buf_b, wsem_b)

        do_elem(i0, 0)
        do_elem(i0 + 1, 1)
        return 0

    lax.fori_loop(0, B_PER_W // 2, pair, 0)
    drain_writes(buf_a, wsem_a)
    drain_writes(buf_b, wsem_b)


@jax.jit
def _bow_broadcast(labels_exp, w_all):
    mesh = plsc.VectorSubcoreMesh(core_axis_name="c", subcore_axis_name="s",
                                  num_cores=NC, num_subcores=NS)
    return pl.kernel(
        _sc_body,
        out_type=jax.ShapeDtypeStruct((BATCH, LM1, VOCAB), jnp.float32),
        mesh=mesh,
        scratch_types=[
            pltpu.VMEM((B_PER_W * NCT,), jnp.int32),
            pltpu.VMEM((NCT, 8, 128), jnp.float32),
            pltpu.VMEM((NCT, 8, 128), jnp.float32),
            pltpu.VMEM((8, VOCAB), jnp.float32),
            pltpu.VMEM((8, VOCAB), jnp.float32),
            pltpu.SemaphoreType.DMA,
            pltpu.SemaphoreType.DMA,
            pltpu.SemaphoreType.DMA,
            pltpu.SemaphoreType.DMA,
        ],
    )(labels_exp, w_all)


def kernel(labels, x, W_word, W_label):
    w_pad = jnp.pad(W_word, ((0, 0), (0, VPAD - VOCAB)))
    w_all = jnp.broadcast_to(
        w_pad.reshape(N_CLS, NCT, 1, 128),
        (N_CLS, NCT, 8, 128)).reshape(N_CLS * NCT, 8, 128)
    labels_exp = (labels.astype(jnp.int32)[:, None] * NCT
                  + jnp.arange(NCT, dtype=jnp.int32)[None, :]).reshape(-1)
    word_logits = _bow_broadcast(labels_exp, jnp.asarray(w_all))
    return (word_logits,)


# speedup vs baseline: 6.9503x; 3.2972x over previous
import jax
import jax.numpy as jnp
from jax import lax
from jax.experimental import pallas as pl
from jax.experimental.pallas import tpu as pltpu
from jax.experimental.pallas import tpu_sc as plsc

BATCH = 1024
LM1 = 49
VOCAB = 1000
VPAD = 1024
N_CLS = 100
CPAD = 112
NSRC = CPAD // 16
NC = 2
NS = 16
NBG = 8
NVG = 4
VSZ = 256
QSZ = 64
NQ = VSZ // QSZ
BSZ = BATCH // NBG
WPIPE = 8

_DNUMS = lax.GatherDimensionNumbers(offset_dims=(), collapsed_slice_dims=(0,),
                                    start_index_map=(0,))


def _sc_body(labels_hbm, wt_hbm, out_hbm, lab_v, wtab, chunk, wsem, csem):
    wid = lax.axis_index("s") * NC + lax.axis_index("c")
    widb = wid % NBG
    widv = wid // NBG
    boff = pl.multiple_of(widb * BSZ, BSZ)
    voff = pl.multiple_of(jnp.where(widv < 3, widv * VSZ, VOCAB - VSZ), 8)

    cp1 = pltpu.async_copy(labels_hbm.at[pl.ds(boff, BSZ)], lab_v, csem)
    cp2 = pltpu.async_copy(wt_hbm.at[pl.ds(voff, VSZ), :], wtab, csem)
    cp1.wait()
    cp2.wait()

    labs = [lab_v[pl.ds(16 * g, 16)] for g in range(BSZ // 16)]
    srcs_of = [lab >> 4 for lab in labs]

    def build_row(v, _):
        cols = [wtab[v, pl.ds(16 * s, 16)] for s in range(NSRC)]
        for g in range(BSZ // 16):
            lab, src_of = labs[g], srcs_of[g]
            idx = (lab & 15)[:, None]
            acc = jnp.zeros((16,), jnp.float32)
            for s in range(NSRC):
                val = lax.gather(cols[s], idx, _DNUMS, (1,),
                                 mode=lax.GatherScatterMode.PROMISE_IN_BOUNDS)
                acc = jnp.where(src_of == s, val, acc)
            chunk[v, pl.ds(16 * g, 16)] = acc
        return 0

    def wait_one():
        pltpu.make_async_copy(
            chunk.at[pl.ds(0, QSZ), :],
            out_hbm.at[0, pl.ds(voff, QSZ), pl.ds(boff, BSZ)], wsem).wait()

    for q in range(NQ):
        lax.fori_loop(q * QSZ, (q + 1) * QSZ, build_row, 0)

        def write_plane(j, _, q=q):
            pltpu.async_copy(
                chunk.at[pl.ds(q * QSZ, QSZ), :],
                out_hbm.at[j, pl.ds(voff + q * QSZ, QSZ), pl.ds(boff, BSZ)],
                wsem)

            @pl.when(q * LM1 + j >= WPIPE)
            def _wait():
                wait_one()

            return 0

        lax.fori_loop(0, LM1, write_plane, 0)

    for _ in range(WPIPE):
        wait_one()


@jax.jit
def _bow_broadcast(labels, w_t):
    mesh = plsc.VectorSubcoreMesh(core_axis_name="c", subcore_axis_name="s",
                                  num_cores=NC, num_subcores=NS)
    return pl.kernel(
        _sc_body,
        out_type=jax.ShapeDtypeStruct((LM1, VOCAB, BATCH), jnp.float32),
        mesh=mesh,
        scratch_types=[
            pltpu.VMEM((BSZ,), jnp.int32),
            pltpu.VMEM((VSZ, CPAD), jnp.float32),
            pltpu.VMEM((VSZ, BSZ), jnp.float32),
            pltpu.SemaphoreType.DMA,
            pltpu.SemaphoreType.DMA,
        ],
    )(labels, w_t)


def kernel(labels, x, W_word, W_label):
    w_t = jnp.pad(W_word.T, ((0, VPAD - VOCAB), (0, CPAD - N_CLS)))
    out_t = _bow_broadcast(labels.astype(jnp.int32), w_t)
    word_logits = jnp.transpose(out_t, (2, 0, 1))
    return (word_logits,)
